# Initial kernel scaffold; baseline (speedup 1.0000x reference)
import jax
import jax.numpy as jnp
from jax.experimental import pallas as pl


def kernel(var_lp_f, con_lp_f, lo_costs, hi_costs, def_mm, edge_lp_f_wo_ss, var_learned_f, con_learned_f, edge_learned_f, edge_index_var_con, batch_index_con, batch_index_edge, norms, params):
    raise NotImplementedError("write your pallas kernel here")


# speedup vs baseline: 2.9742x; 2.9742x over previous
import jax
import jax.numpy as jnp
from jax import lax
from jax.experimental import pallas as pl
from jax.experimental.pallas import tpu as pltpu
from jax.experimental.pallas import tpu_sc as plsc

N = 100000
E = 1600000
NC, NS = 2, 16
NW = NC * NS
PW = E // NW
CG = 2000
NCH = PW // CG
SCW = 80
SROWS = CG // SCW
NZ = N // NS
RN = 2000
RE = 2000
SCALE = 0.25
PREC = lax.Precision.HIGHEST

def _mk_mesh():
    return plsc.VectorSubcoreMesh(core_axis_name="c", subcore_axis_name="s",
                                  num_cores=NC, num_subcores=NS)


_SC_PARAMS = pltpu.CompilerParams(use_tc_tiling_on_sc=False)


def _dot(a, b):
    return jnp.dot(a, b, precision=PREC, preferred_element_type=jnp.float32)


def _gather2(tabA, idxA, tabB, idxB, DA, DB):
    def body(tabA_h, idxA_h, tabB_h, idxB_h, outA_h, outB_h,
             ia_v, ra_v, ib_v, rb_v, sa, sb):
        cid = lax.axis_index("c")
        sid = lax.axis_index("s")
        base = (cid * NS + sid) * PW

        def step(j, carry):
            off = base + j * CG
            pltpu.sync_copy(idxA_h.at[pl.ds(off, CG)], ia_v)
            pltpu.sync_copy(idxB_h.at[pl.ds(off, CG)], ib_v)
            ca = pltpu.async_copy(tabA_h.at[ia_v], ra_v, sa)
            cb = pltpu.async_copy(tabB_h.at[ib_v], rb_v, sb)
            ca.wait()
            cb.wait()
            pltpu.sync_copy(ra_v, outA_h.at[pl.ds(off, CG)])
            pltpu.sync_copy(rb_v, outB_h.at[pl.ds(off, CG)])
            return carry

        lax.fori_loop(0, NCH, step, 0)

    f = pl.kernel(
        body,
        out_type=(jax.ShapeDtypeStruct((E, DA), jnp.float32),
                  jax.ShapeDtypeStruct((E, DB), jnp.float32)),
        mesh=_mk_mesh(),
        scratch_types=[pltpu.VMEM((CG,), jnp.int32),
                       pltpu.VMEM((CG, DA), jnp.float32),
                       pltpu.VMEM((CG,), jnp.int32),
                       pltpu.VMEM((CG, DB), jnp.float32),
                       pltpu.SemaphoreType.DMA,
                       pltpu.SemaphoreType.DMA],
        compiler_params=_SC_PARAMS)
    return f(tabA, idxA, tabB, idxB)


NH = N // NC
HP = NH + 16
PWS = E // NS
NCHS = PWS // CG
NZH = NH // NS
NZP = HP // NS


def _scatter_add(rows, idx2d, zro):
    def body(rows_h, idx_h, z_h, out_h, iv, rv, acc):
        cid = lax.axis_index("c")
        sid = lax.axis_index("s")
        nbase = cid * NH
        pltpu.sync_copy(z_h.at[pl.ds(sid * NZP, NZP)],
                        acc.at[pl.ds(sid * NZP, NZP)])
        plsc.subcore_barrier()
        base = sid * PWS

        def step(j, carry):
            off = base + j * CG
            r0 = off // SCW
            pltpu.sync_copy(idx_h.at[pl.ds(r0, SROWS)], iv)
            pltpu.sync_copy(rows_h.at[pl.ds(off, CG)], rv)

            def remap(r, c3):
                def remap16(v, c4):
                    x = iv[r, pl.ds(v * 16, 16)] - nbase
                    ok = (x >= 0) & (x < NH)
                    iv[r, pl.ds(v * 16, 16)] = jnp.where(ok, x, NH)
                    return c4
                lax.fori_loop(0, SCW // 16, remap16, 0)
                return c3

            lax.fori_loop(0, SROWS, remap, 0)

            def sub(k, c2):
                pltpu.sync_copy(rv.at[pl.ds(k * SCW, SCW)],
                                acc.at[iv.at[k]], add=True)
                return c2

            lax.fori_loop(0, SROWS, sub, 0)
            return carry

        lax.fori_loop(0, NCHS, step, 0)
        plsc.subcore_barrier()
        pltpu.sync_copy(acc.at[pl.ds(sid * NZH, NZH)],
                        out_h.at[pl.ds(nbase + sid * NZH, NZH)])

    f = pl.kernel(
        body,
        out_type=jax.ShapeDtypeStruct((N, 16), jnp.float32),
        mesh=_mk_mesh(),
        scratch_types=[pltpu.VMEM((SROWS, SCW), jnp.int32),
                       pltpu.VMEM((CG, 16), jnp.float32),
                       pltpu.VMEM_SHARED((HP, 16), jnp.float32)],
        compiler_params=_SC_PARAMS)
    return f(rows, idx2d, zro)


def _norm_body(lo_ref, hi_ref, out_ref):
    s = (jnp.sum(jnp.abs(lo_ref[...])) +
         jnp.sum(jnp.abs(hi_ref[...]))) / E + 1e-9
    out_ref[...] = jnp.reshape(s, (1, 1))


def _norm_k(lo2, hi2):
    return pl.pallas_call(
        _norm_body,
        out_shape=jax.ShapeDtypeStruct((1, 1), jnp.float32))(lo2, hi2)


def _lp_norm(clp, n):
    col = lax.broadcasted_iota(jnp.int32, clp.shape, 1)
    return jnp.where(col == 4, clp / n, clp)


def _node_prep_body(cl, clp, vl, vlp, nrm,
                    Wq1, bq1, Ws1, bs1, Wk1, bk1, Wv1, bv1,
                    Wq2, bq2, Ws2, bs2,
                    qc, sc_, kvv, qv, sv):
    n = nrm[0, 0]
    clpn = _lp_norm(clp[...], n)
    cl_ = cl[...]
    vl_ = vl[...]
    vlp_ = vlp[...]
    qc[...] = _dot(cl_, Wq1[0:16]) + _dot(clpn, Wq1[16:21]) + bq1[...]
    sc_[...] = _dot(cl_, Ws1[0:16]) + _dot(clpn, Ws1[16:21]) + bs1[...]
    k = _dot(vl_, Wk1[0:16]) + _dot(vlp_, Wk1[16:18]) + bk1[...]
    v = _dot(vl_, Wv1[0:16]) + _dot(vlp_, Wv1[16:18]) + bv1[...]
    kvv[...] = jnp.concatenate([k, v], axis=1)
    qv[...] = _dot(vl_, Wq2[0:16]) + _dot(vlp_, Wq2[16:18]) + bq2[...]
    sv[...] = _dot(vl_, Ws2[0:16]) + _dot(vlp_, Ws2[16:18]) + bs2[...]


def _node_prep(cl, clp, vl, vlp, nrm, ws):
    g = N // RN
    rows = lambda d: pl.BlockSpec((RN, d), lambda i: (i, 0))
    full = lambda a: pl.BlockSpec(a.shape, lambda i: (0, 0))
    return pl.pallas_call(
        _node_prep_body,
        grid=(g,),
        in_specs=[rows(16), rows(5), rows(16), rows(2),
                  pl.BlockSpec((1, 1), lambda i: (0, 0))] + [full(w) for w in ws],
        out_specs=(rows(16), rows(16), rows(32), rows(16), rows(16)),
        out_shape=(jax.ShapeDtypeStruct((N, 16), jnp.float32),
                   jax.ShapeDtypeStruct((N, 16), jnp.float32),
                   jax.ShapeDtypeStruct((N, 32), jnp.float32),
                   jax.ShapeDtypeStruct((N, 16), jnp.float32),
                   jax.ShapeDtypeStruct((N, 16), jnp.float32)),
    )(cl, clp, vl, vlp, nrm, *ws)


def _ecomb_mm(el, lon, hin, elpn, W):
    return (_dot(el, W[0:16]) + lon * W[16:17] + hin * W[17:18]
            + _dot(elpn, W[18:22]))


def _edge_e_body(el, lo, hi, elp, nrm, We1, be1, We2, be2, e1, e2):
    n = nrm[0, 0]
    lon = lo[...] / n
    hin = hi[...] / n
    col = lax.broadcasted_iota(jnp.int32, (RE, 4), 1)
    elpn = jnp.where(col == 3, elp[...] / n, elp[...])
    el_ = el[...]
    e1[...] = _ecomb_mm(el_, lon, hin, elpn, We1) + be1[...]
    e2[...] = _ecomb_mm(el_, lon, hin, elpn, We2) + be2[...]


def _edge_e(el, lo, hi, elp, nrm, We1, be1, We2, be2):
    g = E // RE
    rows = lambda d: pl.BlockSpec((RE, d), lambda i: (i, 0))
    full = lambda a: pl.BlockSpec(a.shape, lambda i: (0, 0))
    return pl.pallas_call(
        _edge_e_body,
        grid=(g,),
        in_specs=[rows(16), rows(1), rows(1), rows(4),
                  pl.BlockSpec((1, 1), lambda i: (0, 0)),
                  full(We1), full(be1), full(We2), full(be2)],
        out_specs=(rows(16), rows(16)),
        out_shape=(jax.ShapeDtypeStruct((E, 16), jnp.float32),
                   jax.ShapeDtypeStruct((E, 16), jnp.float32)),
    )(el, lo, hi, elp, nrm, We1, be1, We2, be2)


def _alpha_body(qg, kvg, e1, alpha, amax):
    i = pl.program_id(0)
    a = jnp.sum(qg[...] * (kvg[:, 0:16] + e1[...]), axis=1) * SCALE
    alpha[...] = a[:, None]
    m = jnp.max(a)

    @pl.when(i == 0)
    def _():
        amax[...] = jnp.reshape(m, (1, 1))

    @pl.when(i > 0)
    def _():
        amax[...] = jnp.maximum(amax[...], jnp.reshape(m, (1, 1)))


def _alpha_k(qg, kvg, e1):
    g = E // RE
    rows = lambda d: pl.BlockSpec((RE, d), lambda i: (i, 0))
    return pl.pallas_call(
        _alpha_body,
        grid=(g,),
        in_specs=[rows(16), rows(32), rows(16)],
        out_specs=(rows(1), pl.BlockSpec((1, 1), lambda i: (0, 0))),
        out_shape=(jax.ShapeDtypeStruct((E, 1), jnp.float32),
                   jax.ShapeDtypeStruct((1, 1), jnp.float32)),
    )(qg, kvg, e1)


def _contrib_body(alpha, amax, kvg, e1, wrows, exrows):
    ex = jnp.exp(alpha[...] - amax[0, 0])
    wrows[...] = (kvg[:, 16:32] + e1[...]) * ex
    col = lax.broadcasted_iota(jnp.int32, (RE, 16), 1)
    exrows[...] = jnp.where(col == 0, ex,
                            jnp.where(col == 1, 1.0, 0.0))


def _contrib_k(alpha, amax, kvg, e1):
    g = E // RE
    rows = lambda d: pl.BlockSpec((RE, d), lambda i: (i, 0))
    return pl.pallas_call(
        _contrib_body,
        grid=(g,),
        in_specs=[rows(1), pl.BlockSpec((1, 1), lambda i: (0, 0)),
                  rows(32), rows(16)],
        out_specs=(rows(16), rows(16)),
        out_shape=(jax.ShapeDtypeStruct((E, 16), jnp.float32),
                   jax.ShapeDtypeStruct((E, 16), jnp.float32)),
    )(alpha, amax, kvg, e1)


def _combine_body(pw, pe, sk, out):
    w = pw[...]
    eb = pe[...]
    s = eb[:, 0:1]
    cnt = eb[:, 1:2]
    out[...] = jnp.maximum(
        w / (s + 1e-16) / jnp.maximum(cnt, 1.0) + sk[...], 0.0)


def _combine_k(pw, pe, sk):
    g = N // RN
    rows = lambda d: pl.BlockSpec((RN, d), lambda i: (i, 0))
    return pl.pallas_call(
        _combine_body,
        grid=(g,),
        in_specs=[rows(16), rows(16), rows(16)],
        out_specs=rows(16),
        out_shape=jax.ShapeDtypeStruct((N, 16), jnp.float32),
    )(pw, pe, sk)


def _kv2_body(cln, clp, nrm, Wk2, bk2, Wv2, bv2, kv2):
    n = nrm[0, 0]
    clpn = _lp_norm(clp[...], n)
    cl_ = cln[...]
    k = _dot(cl_, Wk2[0:16]) + _dot(clpn, Wk2[16:21]) + bk2[...]
    v = _dot(cl_, Wv2[0:16]) + _dot(clpn, Wv2[16:21]) + bv2[...]
    kv2[...] = jnp.concatenate([k, v], axis=1)


def _kv2_k(cln, clp, nrm, Wk2, bk2, Wv2, bv2):
    g = N // RN
    rows = lambda d: pl.BlockSpec((RN, d), lambda i: (i, 0))
    full = lambda a: pl.BlockSpec(a.shape, lambda i: (0, 0))
    return pl.pallas_call(
        _kv2_body,
        grid=(g,),
        in_specs=[rows(16), rows(5), pl.BlockSpec((1, 1), lambda i: (0, 0)),
                  full(Wk2), full(bk2), full(Wv2), full(bv2)],
        out_specs=rows(32),
        out_shape=jax.ShapeDtypeStruct((N, 32), jnp.float32),
    )(cln, clp, nrm, Wk2, bk2, Wv2, bv2)


def _vfcf_body(vln, vlp, cln, clp, nrm,
               vcW1, vcb1, vcW2, vcb2, ccW1, ccb1, ccW2, ccb2, vf, cf):
    n = nrm[0, 0]
    h = jnp.maximum(_dot(vln[...], vcW1[0:16]) + _dot(vlp[...], vcW1[16:18])
                    + vcb1[...], 0.0)
    vf[...] = jnp.maximum(_dot(h, vcW2[...]) + vcb2[...], 0.0)
    clpn = _lp_norm(clp[...], n)
    h2 = jnp.maximum(_dot(cln[...], ccW1[0:16]) + _dot(clpn, ccW1[16:21])
                     + ccb1[...], 0.0)
    cf[...] = jnp.maximum(_dot(h2, ccW2[...]) + ccb2[...], 0.0)


def _vfcf_k(vln, vlp, cln, clp, nrm, ws):
    g = N // RN
    rows = lambda d: pl.BlockSpec((RN, d), lambda i: (i, 0))
    full = lambda a: pl.BlockSpec(a.shape, lambda i: (0, 0))
    return pl.pallas_call(
        _vfcf_body,
        grid=(g,),
        in_specs=[rows(16), rows(2), rows(16), rows(5),
                  pl.BlockSpec((1, 1), lambda i: (0, 0))] + [full(w) for w in ws],
        out_specs=(rows(16), rows(16)),
        out_shape=(jax.ShapeDtypeStruct((N, 16), jnp.float32),
                   jax.ShapeDtypeStruct((N, 16), jnp.float32)),
    )(vln, vlp, cln, clp, nrm, *ws)


def _final_body(el, lo, hi, elp, vfg, cfg, nrm,
                emW1, emb1, emW2, emb2, pW1, pb1, pW2, pb2,
                loo, hio, eln, ppo):
    n = nrm[0, 0]
    lon = lo[...] / n
    hin = hi[...] / n
    col = lax.broadcasted_iota(jnp.int32, (RE, 4), 1)
    elpn = jnp.where(col == 3, elp[...] / n, elp[...])
    h1 = jnp.maximum(
        _ecomb_mm(el[...], lon, hin, elpn, emW1)
        + _dot(vfg[...], emW1[22:38]) + _dot(cfg[...], emW1[38:54])
        + emb1[...], 0.0)
    e_new = jnp.maximum(_dot(h1, emW2[...]) + emb2[...], 0.0)
    eln[...] = e_new
    t = jnp.maximum(_ecomb_mm(e_new, lon, hin, elpn, pW1) + pb1[...], 0.0)
    pp = 0.1 * (_dot(t, pW2[...]) + pb2[...])
    ppo[...] = pp
    loo[...] = lon + jnp.maximum(pp + 0.005, 0.0)
    hio[...] = hin + jnp.maximum(-pp + 0.005, 0.0)


def _final_k(el, lo, hi, elp, vfg, cfg, nrm, ws):
    g = E // RE
    rows = lambda d: pl.BlockSpec((RE, d), lambda i: (i, 0))
    full = lambda a: pl.BlockSpec(a.shape, lambda i: (0, 0))
    return pl.pallas_call(
        _final_body,
        grid=(g,),
        in_specs=[rows(16), rows(1), rows(1), rows(4), rows(16), rows(16),
                  pl.BlockSpec((1, 1), lambda i: (0, 0))] + [full(w) for w in ws],
        out_specs=(rows(1), rows(1), rows(16), rows(1)),
        out_shape=(jax.ShapeDtypeStruct((E, 1), jnp.float32),
                   jax.ShapeDtypeStruct((E, 1), jnp.float32),
                   jax.ShapeDtypeStruct((E, 16), jnp.float32),
                   jax.ShapeDtypeStruct((E, 1), jnp.float32)),
    )(el, lo, hi, elp, vfg, cfg, nrm, *ws)


def kernel(var_lp_f, con_lp_f, lo_costs, hi_costs, def_mm, edge_lp_f_wo_ss,
           var_learned_f, con_learned_f, edge_learned_f, edge_index_var_con,
           batch_index_con, batch_index_edge, norms, params):
    del def_mm, batch_index_con, batch_index_edge, norms
    tc1 = params['con_updater']
    tc2 = params['var_updater']
    eu = params['eu']
    ppw = params['pp']
    r16 = lambda b: b.reshape(1, 16)

    src = edge_index_var_con[0]
    dst = edge_index_var_con[1]
    lo1 = lo_costs.reshape(E, 1)
    hi1 = hi_costs.reshape(E, 1)

    nrm = _norm_k(lo_costs.reshape(3125, 512), hi_costs.reshape(3125, 512))

    q_con, s_con, kv_var, q_var, s_var = _node_prep(
        con_learned_f, con_lp_f, var_learned_f, var_lp_f, nrm,
        [tc1['Wq'], r16(tc1['bq']), tc1['Ws'], r16(tc1['bs']),
         tc1['Wk'], r16(tc1['bk']), tc1['Wv'], r16(tc1['bv']),
         tc2['Wq'], r16(tc2['bq']), tc2['Ws'], r16(tc2['bs'])])

    e1, e2 = _edge_e(edge_learned_f, lo1, hi1, edge_lp_f_wo_ss, nrm,
                     tc1['We'], r16(tc1['be']), tc2['We'], r16(tc2['be']))

    zro = jnp.zeros((HP, 16), jnp.float32)
    src2d = src.reshape(E // SCW, SCW)
    dst2d = dst.reshape(E // SCW, SCW)

    qg1, kvg1 = _gather2(q_con, dst, kv_var, src, 16, 32)
    alpha1, m1 = _alpha_k(qg1, kvg1, e1)
    w1, x1 = _contrib_k(alpha1, m1, kvg1, e1)
    pw1 = _scatter_add(w1, dst2d, zro)
    pe1 = _scatter_add(x1, dst2d, zro)
    con_ln = _combine_k(pw1, pe1, s_con)

    kv2 = _kv2_k(con_ln, con_lp_f, nrm,
                 tc2['Wk'], r16(tc2['bk']), tc2['Wv'], r16(tc2['bv']))
    qg2, kvg2 = _gather2(q_var, src, kv2, dst, 16, 32)
    alpha2, m2 = _alpha_k(qg2, kvg2, e2)
    w2, x2 = _contrib_k(alpha2, m2, kvg2, e2)
    pw2 = _scatter_add(w2, src2d, zro)
    pe2 = _scatter_add(x2, src2d, zro)
    var_ln = _combine_k(pw2, pe2, s_var)

    vf, cf = _vfcf_k(var_ln, var_lp_f, con_ln, con_lp_f, nrm,
                     [eu['vc_W1'], r16(eu['vc_b1']), eu['vc_W2'], r16(eu['vc_b2']),
                      eu['cc_W1'], r16(eu['cc_b1']), eu['cc_W2'], r16(eu['cc_b2'])])
    vfg, cfg = _gather2(vf, src, cf, dst, 16, 16)
    loo, hio, eln, ppo = _final_k(
        edge_learned_f, lo1, hi1, edge_lp_f_wo_ss, vfg, cfg, nrm,
        [eu['em_W1'], r16(eu['em_b1']), eu['em_W2'], r16(eu['em_b2']),
         ppw['W1'], ppw['b1'].reshape(1, 22), ppw['W2'], ppw['b2'].reshape(1, 1)])

    return (loo.reshape(E), hio.reshape(E), var_ln, con_ln, eln,
            ppo.reshape(E))


# speedup vs baseline: 3.1865x; 1.0714x over previous
import jax
import jax.numpy as jnp
from jax import lax
from jax.experimental import pallas as pl
from jax.experimental.pallas import tpu as pltpu
from jax.experimental.pallas import tpu_sc as plsc

N = 100000
E = 1600000
NC, NS = 2, 16
NW = NC * NS
PW = E // NW
CG = 2000
NCH = PW // CG
SCW = 80
SROWS = CG // SCW
NZ = N // NS
RN = 2000
RE = 3200
RL = RE // 128
GE = E // RE
SCALE = 0.25
PREC = lax.Precision.HIGHEST

def _mk_mesh():
    return plsc.VectorSubcoreMesh(core_axis_name="c", subcore_axis_name="s",
                                  num_cores=NC, num_subcores=NS)


_SC_PARAMS = pltpu.CompilerParams(use_tc_tiling_on_sc=False)


def _dot(a, b):
    return jnp.dot(a, b, precision=PREC, preferred_element_type=jnp.float32)


def _gather2(tabA, idxA, tabB, idxB, DA, DB):
    def body(tabA_h, idxA_h, tabB_h, idxB_h, outA_h, outB_h,
             ia_v, ra_v, ib_v, rb_v, sa, sb):
        cid = lax.axis_index("c")
        sid = lax.axis_index("s")
        base = (cid * NS + sid) * PW

        def step(j, carry):
            off = base + j * CG
            pltpu.sync_copy(idxA_h.at[pl.ds(off, CG)], ia_v)
            pltpu.sync_copy(idxB_h.at[pl.ds(off, CG)], ib_v)
            ca = pltpu.async_copy(tabA_h.at[ia_v], ra_v, sa)
            cb = pltpu.async_copy(tabB_h.at[ib_v], rb_v, sb)
            ca.wait()
            cb.wait()
            pltpu.sync_copy(ra_v, outA_h.at[pl.ds(off, CG)])
            pltpu.sync_copy(rb_v, outB_h.at[pl.ds(off, CG)])
            return carry

        lax.fori_loop(0, NCH, step, 0)

    f = pl.kernel(
        body,
        out_type=(jax.ShapeDtypeStruct((E, DA), jnp.float32),
                  jax.ShapeDtypeStruct((E, DB), jnp.float32)),
        mesh=_mk_mesh(),
        scratch_types=[pltpu.VMEM((CG,), jnp.int32),
                       pltpu.VMEM((CG, DA), jnp.float32),
                       pltpu.VMEM((CG,), jnp.int32),
                       pltpu.VMEM((CG, DB), jnp.float32),
                       pltpu.SemaphoreType.DMA,
                       pltpu.SemaphoreType.DMA],
        compiler_params=_SC_PARAMS)
    return f(tabA, idxA, tabB, idxB)


NH = N // NC
HP = NH + 16
PWS = E // NS
NCHS = PWS // CG
NZH = NH // NS
NZP = HP // NS


def _scatter_add(rows, idx2d, zro):
    def body(rows_h, idx_h, z_h, out_h, iv, rv, acc):
        cid = lax.axis_index("c")
        sid = lax.axis_index("s")
        nbase = cid * NH
        pltpu.sync_copy(z_h.at[pl.ds(sid * NZP, NZP)],
                        acc.at[pl.ds(sid * NZP, NZP)])
        plsc.subcore_barrier()
        base = sid * PWS

        def step(j, carry):
            off = base + j * CG
            r0 = off // SCW
            pltpu.sync_copy(idx_h.at[pl.ds(r0, SROWS)], iv)
            pltpu.sync_copy(rows_h.at[pl.ds(off, CG)], rv)

            def remap(r, c3):
                def remap16(v, c4):
                    x = iv[r, pl.ds(v * 16, 16)] - nbase
                    ok = (x >= 0) & (x < NH)
                    iv[r, pl.ds(v * 16, 16)] = jnp.where(ok, x, NH)
                    return c4
                lax.fori_loop(0, SCW // 16, remap16, 0)
                return c3

            lax.fori_loop(0, SROWS, remap, 0)

            def sub(k, c2):
                pltpu.sync_copy(rv.at[pl.ds(k * SCW, SCW)],
                                acc.at[iv.at[k]], add=True)
                return c2

            lax.fori_loop(0, SROWS, sub, 0)
            return carry

        lax.fori_loop(0, NCHS, step, 0)
        plsc.subcore_barrier()
        pltpu.sync_copy(acc.at[pl.ds(sid * NZH, NZH)],
                        out_h.at[pl.ds(nbase + sid * NZH, NZH)])

    f = pl.kernel(
        body,
        out_type=jax.ShapeDtypeStruct((N, 16), jnp.float32),
        mesh=_mk_mesh(),
        scratch_types=[pltpu.VMEM((SROWS, SCW), jnp.int32),
                       pltpu.VMEM((CG, 16), jnp.float32),
                       pltpu.VMEM_SHARED((HP, 16), jnp.float32)],
        compiler_params=_SC_PARAMS)
    return f(rows, idx2d, zro)


def _norm_body(lo_ref, hi_ref, out_ref):
    s = (jnp.sum(jnp.abs(lo_ref[...])) +
         jnp.sum(jnp.abs(hi_ref[...]))) / E + 1e-9
    out_ref[...] = jnp.reshape(s, (1, 1))


def _norm_k(lo2, hi2):
    return pl.pallas_call(
        _norm_body,
        out_shape=jax.ShapeDtypeStruct((1, 1), jnp.float32))(lo2, hi2)


def _lp_norm(clp, n):
    col = lax.broadcasted_iota(jnp.int32, clp.shape, 1)
    return jnp.where(col == 4, clp / n, clp)


def _node_prep_body(cl, clp, vl, vlp, nrm,
                    Wq1, bq1, Ws1, bs1, Wk1, bk1, Wv1, bv1,
                    Wq2, bq2, Ws2, bs2,
                    qc, sc_, kvv, qv, sv):
    n = nrm[0, 0]
    clpn = _lp_norm(clp[...], n)
    cl_ = cl[...]
    vl_ = vl[...]
    vlp_ = vlp[...]
    qc[...] = _dot(cl_, Wq1[0:16]) + _dot(clpn, Wq1[16:21]) + bq1[...]
    sc_[...] = _dot(cl_, Ws1[0:16]) + _dot(clpn, Ws1[16:21]) + bs1[...]
    k = _dot(vl_, Wk1[0:16]) + _dot(vlp_, Wk1[16:18]) + bk1[...]
    v = _dot(vl_, Wv1[0:16]) + _dot(vlp_, Wv1[16:18]) + bv1[...]
    kvv[...] = jnp.concatenate([k, v], axis=1)
    qv[...] = _dot(vl_, Wq2[0:16]) + _dot(vlp_, Wq2[16:18]) + bq2[...]
    sv[...] = _dot(vl_, Ws2[0:16]) + _dot(vlp_, Ws2[16:18]) + bs2[...]


def _node_prep(cl, clp, vl, vlp, nrm, ws):
    g = N // RN
    rows = lambda d: pl.BlockSpec((RN, d), lambda i: (i, 0))
    full = lambda a: pl.BlockSpec(a.shape, lambda i: (0, 0))
    return pl.pallas_call(
        _node_prep_body,
        grid=(g,),
        in_specs=[rows(16), rows(5), rows(16), rows(2),
                  pl.BlockSpec((1, 1), lambda i: (0, 0))] + [full(w) for w in ws],
        out_specs=(rows(16), rows(16), rows(32), rows(16), rows(16)),
        out_shape=(jax.ShapeDtypeStruct((N, 16), jnp.float32),
                   jax.ShapeDtypeStruct((N, 16), jnp.float32),
                   jax.ShapeDtypeStruct((N, 32), jnp.float32),
                   jax.ShapeDtypeStruct((N, 16), jnp.float32),
                   jax.ShapeDtypeStruct((N, 16), jnp.float32)),
    )(cl, clp, vl, vlp, nrm, *ws)


def _ecomb_mm(el, lon, hin, elpn, W):
    return (_dot(el, W[0:16]) + lon * W[16:17] + hin * W[17:18]
            + _dot(elpn, W[18:22]))


def _edge_e_body(el, sc8, nrm, We1, be1, We2, be2, e1, e2):
    n = nrm[0, 0]
    b = sc8[...]
    lon = b[:, 4:5] / n
    hin = b[:, 5:6] / n
    col = lax.broadcasted_iota(jnp.int32, (RE, 4), 1)
    raw = b[:, 0:4]
    elpn = jnp.where(col == 3, raw / n, raw)
    el_ = el[...]
    e1[...] = _ecomb_mm(el_, lon, hin, elpn, We1) + be1[...]
    e2[...] = _ecomb_mm(el_, lon, hin, elpn, We2) + be2[...]


def _edge_e(el, sc8, nrm, We1, be1, We2, be2):
    rows = lambda d: pl.BlockSpec((RE, d), lambda i: (i, 0))
    full = lambda a: pl.BlockSpec(a.shape, lambda i: (0, 0))
    return pl.pallas_call(
        _edge_e_body,
        grid=(GE,),
        in_specs=[rows(16), rows(8),
                  pl.BlockSpec((1, 1), lambda i: (0, 0)),
                  full(We1), full(be1), full(We2), full(be2)],
        out_specs=(rows(16), rows(16)),
        out_shape=(jax.ShapeDtypeStruct((E, 16), jnp.float32),
                   jax.ShapeDtypeStruct((E, 16), jnp.float32)),
    )(el, sc8, nrm, We1, be1, We2, be2)


def _att_body(qg, kvg, e1, wrows, exrows):
    kv = kvg[...]
    e = e1[...]
    a = jnp.sum(qg[...] * (kv[:, 0:16] + e), axis=1, keepdims=True) * SCALE
    ex = jnp.exp(a)
    wrows[...] = (kv[:, 16:32] + e) * ex
    col = lax.broadcasted_iota(jnp.int32, (RE, 16), 1)
    exrows[...] = jnp.where(col == 0, ex,
                            jnp.where(col == 1, 1.0, 0.0))


def _att_k(qg, kvg, e1):
    g = E // RE
    rows = lambda d: pl.BlockSpec((RE, d), lambda i: (i, 0))
    return pl.pallas_call(
        _att_body,
        grid=(g,),
        in_specs=[rows(16), rows(32), rows(16)],
        out_specs=(rows(16), rows(16)),
        out_shape=(jax.ShapeDtypeStruct((E, 16), jnp.float32),
                   jax.ShapeDtypeStruct((E, 16), jnp.float32)),
    )(qg, kvg, e1)


def _combine_body(pw, pe, sk, out):
    w = pw[...]
    eb = pe[...]
    s = eb[:, 0:1]
    cnt = eb[:, 1:2]
    out[...] = jnp.maximum(
        w / (s + 1e-16) / jnp.maximum(cnt, 1.0) + sk[...], 0.0)


def _combine_k(pw, pe, sk):
    g = N // RN
    rows = lambda d: pl.BlockSpec((RN, d), lambda i: (i, 0))
    return pl.pallas_call(
        _combine_body,
        grid=(g,),
        in_specs=[rows(16), rows(16), rows(16)],
        out_specs=rows(16),
        out_shape=jax.ShapeDtypeStruct((N, 16), jnp.float32),
    )(pw, pe, sk)


def _kv2_body(cln, clp, nrm, Wk2, bk2, Wv2, bv2, kv2):
    n = nrm[0, 0]
    clpn = _lp_norm(clp[...], n)
    cl_ = cln[...]
    k = _dot(cl_, Wk2[0:16]) + _dot(clpn, Wk2[16:21]) + bk2[...]
    v = _dot(cl_, Wv2[0:16]) + _dot(clpn, Wv2[16:21]) + bv2[...]
    kv2[...] = jnp.concatenate([k, v], axis=1)


def _kv2_k(cln, clp, nrm, Wk2, bk2, Wv2, bv2):
    g = N // RN
    rows = lambda d: pl.BlockSpec((RN, d), lambda i: (i, 0))
    full = lambda a: pl.BlockSpec(a.shape, lambda i: (0, 0))
    return pl.pallas_call(
        _kv2_body,
        grid=(g,),
        in_specs=[rows(16), rows(5), pl.BlockSpec((1, 1), lambda i: (0, 0)),
                  full(Wk2), full(bk2), full(Wv2), full(bv2)],
        out_specs=rows(32),
        out_shape=jax.ShapeDtypeStruct((N, 32), jnp.float32),
    )(cln, clp, nrm, Wk2, bk2, Wv2, bv2)


def _vfcf_body(vln, vlp, cln, clp, nrm,
               vcW1, vcb1, vcW2, vcb2, ccW1, ccb1, ccW2, ccb2, vf, cf):
    n = nrm[0, 0]
    h = jnp.maximum(_dot(vln[...], vcW1[0:16]) + _dot(vlp[...], vcW1[16:18])
                    + vcb1[...], 0.0)
    vf[...] = jnp.maximum(_dot(h, vcW2[...]) + vcb2[...], 0.0)
    clpn = _lp_norm(clp[...], n)
    h2 = jnp.maximum(_dot(cln[...], ccW1[0:16]) + _dot(clpn, ccW1[16:21])
                     + ccb1[...], 0.0)
    cf[...] = jnp.maximum(_dot(h2, ccW2[...]) + ccb2[...], 0.0)


def _vfcf_k(vln, vlp, cln, clp, nrm, ws):
    g = N // RN
    rows = lambda d: pl.BlockSpec((RN, d), lambda i: (i, 0))
    full = lambda a: pl.BlockSpec(a.shape, lambda i: (0, 0))
    return pl.pallas_call(
        _vfcf_body,
        grid=(g,),
        in_specs=[rows(16), rows(2), rows(16), rows(5),
                  pl.BlockSpec((1, 1), lambda i: (0, 0))] + [full(w) for w in ws],
        out_specs=(rows(16), rows(16)),
        out_shape=(jax.ShapeDtypeStruct((N, 16), jnp.float32),
                   jax.ShapeDtypeStruct((N, 16), jnp.float32)),
    )(vln, vlp, cln, clp, nrm, *ws)


def _final_body(el, sc8, vfg, cfg, nrm,
                emW1, emb1, emW2, emb2, pW1, pb1, pW2, pb2,
                outb, eln):
    n = nrm[0, 0]
    b = sc8[...]
    lon = b[:, 4:5] / n
    hin = b[:, 5:6] / n
    col = lax.broadcasted_iota(jnp.int32, (RE, 4), 1)
    raw = b[:, 0:4]
    elpn = jnp.where(col == 3, raw / n, raw)
    h1 = jnp.maximum(
        _ecomb_mm(el[...], lon, hin, elpn, emW1)
        + _dot(vfg[...], emW1[22:38]) + _dot(cfg[...], emW1[38:54])
        + emb1[...], 0.0)
    e_new = jnp.maximum(_dot(h1, emW2[...]) + emb2[...], 0.0)
    eln[...] = e_new
    t = jnp.maximum(_ecomb_mm(e_new, lon, hin, elpn, pW1) + pb1[...], 0.0)
    pp = 0.1 * (_dot(t, pW2[...]) + pb2[...])
    loo = lon + jnp.maximum(pp + 0.005, 0.0)
    hio = hin + jnp.maximum(-pp + 0.005, 0.0)
    col8 = lax.broadcasted_iota(jnp.int32, (RE, 8), 1)
    outb[...] = jnp.where(col8 == 0, loo,
                          jnp.where(col8 == 1, hio,
                                    jnp.where(col8 == 2, pp, 0.0)))


def _final_k(el, sc8, vfg, cfg, nrm, ws):
    rows = lambda d: pl.BlockSpec((RE, d), lambda i: (i, 0))
    full = lambda a: pl.BlockSpec(a.shape, lambda i: (0, 0))
    return pl.pallas_call(
        _final_body,
        grid=(GE,),
        in_specs=[rows(16), rows(8), rows(16), rows(16),
                  pl.BlockSpec((1, 1), lambda i: (0, 0))] + [full(w) for w in ws],
        out_specs=(rows(8), rows(16)),
        out_shape=(jax.ShapeDtypeStruct((E, 8), jnp.float32),
                   jax.ShapeDtypeStruct((E, 16), jnp.float32)),
    )(el, sc8, vfg, cfg, nrm, *ws)


def kernel(var_lp_f, con_lp_f, lo_costs, hi_costs, def_mm, edge_lp_f_wo_ss,
           var_learned_f, con_learned_f, edge_learned_f, edge_index_var_con,
           batch_index_con, batch_index_edge, norms, params):
    del def_mm, batch_index_con, batch_index_edge, norms
    tc1 = params['con_updater']
    tc2 = params['var_updater']
    eu = params['eu']
    ppw = params['pp']
    r16 = lambda b: b.reshape(1, 16)

    src = edge_index_var_con[0]
    dst = edge_index_var_con[1]
    sc8 = jnp.concatenate(
        [edge_lp_f_wo_ss, lo_costs[:, None], hi_costs[:, None],
         jnp.zeros((E, 2), jnp.float32)], axis=1)

    nrm = _norm_k(lo_costs.reshape(3125, 512), hi_costs.reshape(3125, 512))

    q_con, s_con, kv_var, q_var, s_var = _node_prep(
        con_learned_f, con_lp_f, var_learned_f, var_lp_f, nrm,
        [tc1['Wq'], r16(tc1['bq']), tc1['Ws'], r16(tc1['bs']),
         tc1['Wk'], r16(tc1['bk']), tc1['Wv'], r16(tc1['bv']),
         tc2['Wq'], r16(tc2['bq']), tc2['Ws'], r16(tc2['bs'])])

    e1, e2 = _edge_e(edge_learned_f, sc8, nrm,
                     tc1['We'], r16(tc1['be']), tc2['We'], r16(tc2['be']))

    zro = jnp.zeros((HP, 16), jnp.float32)
    src2d = src.reshape(E // SCW, SCW)
    dst2d = dst.reshape(E // SCW, SCW)

    qg1, kvg1 = _gather2(q_con, dst, kv_var, src, 16, 32)
    w1, x1 = _att_k(qg1, kvg1, e1)
    pw1 = _scatter_add(w1, dst2d, zro)
    pe1 = _scatter_add(x1, dst2d, zro)
    con_ln = _combine_k(pw1, pe1, s_con)

    kv2 = _kv2_k(con_ln, con_lp_f, nrm,
                 tc2['Wk'], r16(tc2['bk']), tc2['Wv'], r16(tc2['bv']))
    qg2, kvg2 = _gather2(q_var, src, kv2, dst, 16, 32)
    w2, x2 = _att_k(qg2, kvg2, e2)
    pw2 = _scatter_add(w2, src2d, zro)
    pe2 = _scatter_add(x2, src2d, zro)
    var_ln = _combine_k(pw2, pe2, s_var)

    vf, cf = _vfcf_k(var_ln, var_lp_f, con_ln, con_lp_f, nrm,
                     [eu['vc_W1'], r16(eu['vc_b1']), eu['vc_W2'], r16(eu['vc_b2']),
                      eu['cc_W1'], r16(eu['cc_b1']), eu['cc_W2'], r16(eu['cc_b2'])])
    vfg, cfg = _gather2(vf, src, cf, dst, 16, 16)
    outb, eln = _final_k(
        edge_learned_f, sc8, vfg, cfg, nrm,
        [eu['em_W1'], r16(eu['em_b1']), eu['em_W2'], r16(eu['em_b2']),
         ppw['W1'], ppw['b1'].reshape(1, 22), ppw['W2'], ppw['b2'].reshape(1, 1)])

    return (outb[:, 0], outb[:, 1], var_ln, con_ln, eln, outb[:, 2])


# speedup vs baseline: 5.0815x; 1.5947x over previous
import jax
import jax.numpy as jnp
from jax import lax
from jax.experimental import pallas as pl
from jax.experimental.pallas import tpu as pltpu
from jax.experimental.pallas import tpu_sc as plsc

N = 100000
E = 1600000
NC, NS = 2, 16
NW = NC * NS
PW = E // NW
CG = 2000
NCH = PW // CG
SCW = 80
SROWS = CG // SCW
NZ = N // NS
RN = 2000
RE = 3200
RL = RE // 128
GE = E // RE
SCALE = 0.25
PREC = lax.Precision.DEFAULT

def _mk_mesh():
    return plsc.VectorSubcoreMesh(core_axis_name="c", subcore_axis_name="s",
                                  num_cores=NC, num_subcores=NS)


_SC_PARAMS = pltpu.CompilerParams(use_tc_tiling_on_sc=False)


def _dot(a, b):
    return jnp.dot(a, b, precision=PREC, preferred_element_type=jnp.float32)


def _gather2(tabA, idxA, tabB, idxB, DA, DB):
    def body(tabA_h, idxA_h, tabB_h, idxB_h, outA_h, outB_h,
             ia_v, ra_v, ib_v, rb_v, sa, sb):
        cid = lax.axis_index("c")
        sid = lax.axis_index("s")
        base = (cid * NS + sid) * PW

        def step(j, carry):
            off = base + j * CG
            pltpu.sync_copy(idxA_h.at[pl.ds(off, CG)], ia_v)
            pltpu.sync_copy(idxB_h.at[pl.ds(off, CG)], ib_v)
            ca = pltpu.async_copy(tabA_h.at[ia_v], ra_v, sa)
            cb = pltpu.async_copy(tabB_h.at[ib_v], rb_v, sb)
            ca.wait()
            cb.wait()
            pltpu.sync_copy(ra_v, outA_h.at[pl.ds(off, CG)])
            pltpu.sync_copy(rb_v, outB_h.at[pl.ds(off, CG)])
            return carry

        lax.fori_loop(0, NCH, step, 0)

    f = pl.kernel(
        body,
        out_type=(jax.ShapeDtypeStruct((E, DA), jnp.float32),
                  jax.ShapeDtypeStruct((E, DB), jnp.float32)),
        mesh=_mk_mesh(),
        scratch_types=[pltpu.VMEM((CG,), jnp.int32),
                       pltpu.VMEM((CG, DA), jnp.float32),
                       pltpu.VMEM((CG,), jnp.int32),
                       pltpu.VMEM((CG, DB), jnp.float32),
                       pltpu.SemaphoreType.DMA,
                       pltpu.SemaphoreType.DMA],
        compiler_params=_SC_PARAMS)
    return f(tabA, idxA, tabB, idxB)


NH = N // NC
HP = NH + 16
PWS = E // NS
NCHS = PWS // CG
NZH = NH // NS
NZP = HP // NS


def _scatter_add(rows, idx2d, zro):
    def body(rows_h, idx_h, z_h, out_h, iv, rv, acc):
        cid = lax.axis_index("c")
        sid = lax.axis_index("s")
        nbase = cid * NH
        pltpu.sync_copy(z_h.at[pl.ds(sid * NZP, NZP)],
                        acc.at[pl.ds(sid * NZP, NZP)])
        plsc.subcore_barrier()
        base = sid * PWS

        def step(j, carry):
            off = base + j * CG
            r0 = off // SCW
            pltpu.sync_copy(idx_h.at[pl.ds(r0, SROWS)], iv)
            pltpu.sync_copy(rows_h.at[pl.ds(off, CG)], rv)

            def remap(r, c3):
                def remap16(v, c4):
                    x = iv[r, pl.ds(v * 16, 16)] - nbase
                    ok = (x >= 0) & (x < NH)
                    iv[r, pl.ds(v * 16, 16)] = jnp.where(ok, x, NH)
                    return c4
                lax.fori_loop(0, SCW // 16, remap16, 0)
                return c3

            lax.fori_loop(0, SROWS, remap, 0)

            def sub(k, c2):
                pltpu.sync_copy(rv.at[pl.ds(k * SCW, SCW)],
                                acc.at[iv.at[k]], add=True)
                return c2

            lax.fori_loop(0, SROWS, sub, 0)
            return carry

        lax.fori_loop(0, NCHS, step, 0)
        plsc.subcore_barrier()
        pltpu.sync_copy(acc.at[pl.ds(sid * NZH, NZH)],
                        out_h.at[pl.ds(nbase + sid * NZH, NZH)])

    f = pl.kernel(
        body,
        out_type=jax.ShapeDtypeStruct((N, 16), jnp.float32),
        mesh=_mk_mesh(),
        scratch_types=[pltpu.VMEM((SROWS, SCW), jnp.int32),
                       pltpu.VMEM((CG, 16), jnp.float32),
                       pltpu.VMEM_SHARED((HP, 16), jnp.float32)],
        compiler_params=_SC_PARAMS)
    return f(rows, idx2d, zro)


def _norm_body(lo_ref, hi_ref, out_ref):
    s = (jnp.sum(jnp.abs(lo_ref[...])) +
         jnp.sum(jnp.abs(hi_ref[...]))) / E + 1e-9
    out_ref[...] = jnp.reshape(s, (1, 1))


def _norm_k(lo2, hi2):
    return pl.pallas_call(
        _norm_body,
        out_shape=jax.ShapeDtypeStruct((1, 1), jnp.float32))(lo2, hi2)


def _lp_norm(clp, n):
    col = lax.broadcasted_iota(jnp.int32, clp.shape, 1)
    return jnp.where(col == 4, clp / n, clp)


def _node_prep_body(cl, clp, vl, vlp, nrm,
                    Wq1, bq1, Ws1, bs1, Wk1, bk1, Wv1, bv1,
                    Wq2, bq2, Ws2, bs2,
                    qc, sc_, kvv, qv, sv):
    n = nrm[0, 0]
    clpn = _lp_norm(clp[...], n)
    cl_ = cl[...]
    vl_ = vl[...]
    vlp_ = vlp[...]
    qc[...] = _dot(cl_, Wq1[0:16]) + _dot(clpn, Wq1[16:21]) + bq1[...]
    sc_[...] = _dot(cl_, Ws1[0:16]) + _dot(clpn, Ws1[16:21]) + bs1[...]
    k = _dot(vl_, Wk1[0:16]) + _dot(vlp_, Wk1[16:18]) + bk1[...]
    v = _dot(vl_, Wv1[0:16]) + _dot(vlp_, Wv1[16:18]) + bv1[...]
    kvv[...] = jnp.concatenate([k, v], axis=1)
    qv[...] = _dot(vl_, Wq2[0:16]) + _dot(vlp_, Wq2[16:18]) + bq2[...]
    sv[...] = _dot(vl_, Ws2[0:16]) + _dot(vlp_, Ws2[16:18]) + bs2[...]


def _node_prep(cl, clp, vl, vlp, nrm, ws):
    g = N // RN
    rows = lambda d: pl.BlockSpec((RN, d), lambda i: (i, 0))
    full = lambda a: pl.BlockSpec(a.shape, lambda i: (0, 0))
    return pl.pallas_call(
        _node_prep_body,
        grid=(g,),
        in_specs=[rows(16), rows(5), rows(16), rows(2),
                  pl.BlockSpec((1, 1), lambda i: (0, 0))] + [full(w) for w in ws],
        out_specs=(rows(16), rows(16), rows(32), rows(16), rows(16)),
        out_shape=(jax.ShapeDtypeStruct((N, 16), jnp.float32),
                   jax.ShapeDtypeStruct((N, 16), jnp.float32),
                   jax.ShapeDtypeStruct((N, 32), jnp.float32),
                   jax.ShapeDtypeStruct((N, 16), jnp.float32),
                   jax.ShapeDtypeStruct((N, 16), jnp.float32)),
    )(cl, clp, vl, vlp, nrm, *ws)


def _ecomb_mm(el, lon, hin, elpn, W):
    return (_dot(el, W[0:16]) + lon * W[16:17] + hin * W[17:18]
            + _dot(elpn, W[18:22]))


def _edge_e_body(el, sc8, nrm, We1, be1, We2, be2, e1, e2):
    n = nrm[0, 0]
    b = sc8[...]
    lon = b[:, 4:5] / n
    hin = b[:, 5:6] / n
    col = lax.broadcasted_iota(jnp.int32, (RE, 4), 1)
    raw = b[:, 0:4]
    elpn = jnp.where(col == 3, raw / n, raw)
    el_ = el[...]
    e1[...] = _ecomb_mm(el_, lon, hin, elpn, We1) + be1[...]
    e2[...] = _ecomb_mm(el_, lon, hin, elpn, We2) + be2[...]


def _edge_e(el, sc8, nrm, We1, be1, We2, be2):
    rows = lambda d: pl.BlockSpec((RE, d), lambda i: (i, 0))
    full = lambda a: pl.BlockSpec(a.shape, lambda i: (0, 0))
    return pl.pallas_call(
        _edge_e_body,
        grid=(GE,),
        in_specs=[rows(16), rows(8),
                  pl.BlockSpec((1, 1), lambda i: (0, 0)),
                  full(We1), full(be1), full(We2), full(be2)],
        out_specs=(rows(16), rows(16)),
        out_shape=(jax.ShapeDtypeStruct((E, 16), jnp.float32),
                   jax.ShapeDtypeStruct((E, 16), jnp.float32)),
    )(el, sc8, nrm, We1, be1, We2, be2)


def _att_body(qg, kvg, e1, wrows, exrows):
    kv = kvg[...]
    e = e1[...]
    a = jnp.sum(qg[...] * (kv[:, 0:16] + e), axis=1, keepdims=True) * SCALE
    ex = jnp.exp(a)
    wrows[...] = (kv[:, 16:32] + e) * ex
    col = lax.broadcasted_iota(jnp.int32, (RE, 16), 1)
    exrows[...] = jnp.where(col == 0, ex,
                            jnp.where(col == 1, 1.0, 0.0))


def _att_k(qg, kvg, e1):
    g = E // RE
    rows = lambda d: pl.BlockSpec((RE, d), lambda i: (i, 0))
    return pl.pallas_call(
        _att_body,
        grid=(g,),
        in_specs=[rows(16), rows(32), rows(16)],
        out_specs=(rows(16), rows(16)),
        out_shape=(jax.ShapeDtypeStruct((E, 16), jnp.float32),
                   jax.ShapeDtypeStruct((E, 16), jnp.float32)),
    )(qg, kvg, e1)


def _combine_body(pw, pe, sk, out):
    w = pw[...]
    eb = pe[...]
    s = eb[:, 0:1]
    cnt = eb[:, 1:2]
    out[...] = jnp.maximum(
        w / (s + 1e-16) / jnp.maximum(cnt, 1.0) + sk[...], 0.0)


def _combine_k(pw, pe, sk):
    g = N // RN
    rows = lambda d: pl.BlockSpec((RN, d), lambda i: (i, 0))
    return pl.pallas_call(
        _combine_body,
        grid=(g,),
        in_specs=[rows(16), rows(16), rows(16)],
        out_specs=rows(16),
        out_shape=jax.ShapeDtypeStruct((N, 16), jnp.float32),
    )(pw, pe, sk)


def _kv2_body(cln, clp, nrm, Wk2, bk2, Wv2, bv2, kv2):
    n = nrm[0, 0]
    clpn = _lp_norm(clp[...], n)
    cl_ = cln[...]
    k = _dot(cl_, Wk2[0:16]) + _dot(clpn, Wk2[16:21]) + bk2[...]
    v = _dot(cl_, Wv2[0:16]) + _dot(clpn, Wv2[16:21]) + bv2[...]
    kv2[...] = jnp.concatenate([k, v], axis=1)


def _kv2_k(cln, clp, nrm, Wk2, bk2, Wv2, bv2):
    g = N // RN
    rows = lambda d: pl.BlockSpec((RN, d), lambda i: (i, 0))
    full = lambda a: pl.BlockSpec(a.shape, lambda i: (0, 0))
    return pl.pallas_call(
        _kv2_body,
        grid=(g,),
        in_specs=[rows(16), rows(5), pl.BlockSpec((1, 1), lambda i: (0, 0)),
                  full(Wk2), full(bk2), full(Wv2), full(bv2)],
        out_specs=rows(32),
        out_shape=jax.ShapeDtypeStruct((N, 32), jnp.float32),
    )(cln, clp, nrm, Wk2, bk2, Wv2, bv2)


def _vfcf_body(vln, vlp, cln, clp, nrm,
               vcW1, vcb1, vcW2, vcb2, ccW1, ccb1, ccW2, ccb2, vf, cf):
    n = nrm[0, 0]
    h = jnp.maximum(_dot(vln[...], vcW1[0:16]) + _dot(vlp[...], vcW1[16:18])
                    + vcb1[...], 0.0)
    vf[...] = jnp.maximum(_dot(h, vcW2[...]) + vcb2[...], 0.0)
    clpn = _lp_norm(clp[...], n)
    h2 = jnp.maximum(_dot(cln[...], ccW1[0:16]) + _dot(clpn, ccW1[16:21])
                     + ccb1[...], 0.0)
    cf[...] = jnp.maximum(_dot(h2, ccW2[...]) + ccb2[...], 0.0)


def _vfcf_k(vln, vlp, cln, clp, nrm, ws):
    g = N // RN
    rows = lambda d: pl.BlockSpec((RN, d), lambda i: (i, 0))
    full = lambda a: pl.BlockSpec(a.shape, lambda i: (0, 0))
    return pl.pallas_call(
        _vfcf_body,
        grid=(g,),
        in_specs=[rows(16), rows(2), rows(16), rows(5),
                  pl.BlockSpec((1, 1), lambda i: (0, 0))] + [full(w) for w in ws],
        out_specs=(rows(16), rows(16)),
        out_shape=(jax.ShapeDtypeStruct((N, 16), jnp.float32),
                   jax.ShapeDtypeStruct((N, 16), jnp.float32)),
    )(vln, vlp, cln, clp, nrm, *ws)


def _final_body(el, sc8, vfg, cfg, nrm,
                emW1, emb1, emW2, emb2, pW1, pb1, pW2, pb2,
                outb, eln):
    n = nrm[0, 0]
    b = sc8[...]
    lon = b[:, 4:5] / n
    hin = b[:, 5:6] / n
    col = lax.broadcasted_iota(jnp.int32, (RE, 4), 1)
    raw = b[:, 0:4]
    elpn = jnp.where(col == 3, raw / n, raw)
    h1 = jnp.maximum(
        _ecomb_mm(el[...], lon, hin, elpn, emW1)
        + _dot(vfg[...], emW1[22:38]) + _dot(cfg[...], emW1[38:54])
        + emb1[...], 0.0)
    e_new = jnp.maximum(_dot(h1, emW2[...]) + emb2[...], 0.0)
    eln[...] = e_new
    t = jnp.maximum(_ecomb_mm(e_new, lon, hin, elpn, pW1) + pb1[...], 0.0)
    pp = 0.1 * (_dot(t, pW2[...]) + pb2[...])
    loo = lon + jnp.maximum(pp + 0.005, 0.0)
    hio = hin + jnp.maximum(-pp + 0.005, 0.0)
    col8 = lax.broadcasted_iota(jnp.int32, (RE, 8), 1)
    outb[...] = jnp.where(col8 == 0, loo,
                          jnp.where(col8 == 1, hio,
                                    jnp.where(col8 == 2, pp, 0.0)))


def _final_k(el, sc8, vfg, cfg, nrm, ws):
    rows = lambda d: pl.BlockSpec((RE, d), lambda i: (i, 0))
    full = lambda a: pl.BlockSpec(a.shape, lambda i: (0, 0))
    return pl.pallas_call(
        _final_body,
        grid=(GE,),
        in_specs=[rows(16), rows(8), rows(16), rows(16),
                  pl.BlockSpec((1, 1), lambda i: (0, 0))] + [full(w) for w in ws],
        out_specs=(rows(8), rows(16)),
        out_shape=(jax.ShapeDtypeStruct((E, 8), jnp.float32),
                   jax.ShapeDtypeStruct((E, 16), jnp.float32)),
    )(el, sc8, vfg, cfg, nrm, *ws)


def kernel(var_lp_f, con_lp_f, lo_costs, hi_costs, def_mm, edge_lp_f_wo_ss,
           var_learned_f, con_learned_f, edge_learned_f, edge_index_var_con,
           batch_index_con, batch_index_edge, norms, params):
    del def_mm, batch_index_con, batch_index_edge, norms
    tc1 = params['con_updater']
    tc2 = params['var_updater']
    eu = params['eu']
    ppw = params['pp']
    r16 = lambda b: b.reshape(1, 16)

    src = edge_index_var_con[0]
    dst = edge_index_var_con[1]
    sc8 = jnp.concatenate(
        [edge_lp_f_wo_ss, lo_costs[:, None], hi_costs[:, None],
         jnp.zeros((E, 2), jnp.float32)], axis=1)

    nrm = _norm_k(lo_costs.reshape(3125, 512), hi_costs.reshape(3125, 512))

    q_con, s_con, kv_var, q_var, s_var = _node_prep(
        con_learned_f, con_lp_f, var_learned_f, var_lp_f, nrm,
        [tc1['Wq'], r16(tc1['bq']), tc1['Ws'], r16(tc1['bs']),
         tc1['Wk'], r16(tc1['bk']), tc1['Wv'], r16(tc1['bv']),
         tc2['Wq'], r16(tc2['bq']), tc2['Ws'], r16(tc2['bs'])])

    e1, e2 = _edge_e(edge_learned_f, sc8, nrm,
                     tc1['We'], r16(tc1['be']), tc2['We'], r16(tc2['be']))

    zro = jnp.zeros((HP, 16), jnp.float32)
    src2d = src.reshape(E // SCW, SCW)
    dst2d = dst.reshape(E // SCW, SCW)

    qg1, kvg1 = _gather2(q_con, dst, kv_var, src, 16, 32)
    w1, x1 = _att_k(qg1, kvg1, e1)
    pw1 = _scatter_add(w1, dst2d, zro)
    pe1 = _scatter_add(x1, dst2d, zro)
    con_ln = _combine_k(pw1, pe1, s_con)

    kv2 = _kv2_k(con_ln, con_lp_f, nrm,
                 tc2['Wk'], r16(tc2['bk']), tc2['Wv'], r16(tc2['bv']))
    qg2, kvg2 = _gather2(q_var, src, kv2, dst, 16, 32)
    w2, x2 = _att_k(qg2, kvg2, e2)
    pw2 = _scatter_add(w2, src2d, zro)
    pe2 = _scatter_add(x2, src2d, zro)
    var_ln = _combine_k(pw2, pe2, s_var)

    vf, cf = _vfcf_k(var_ln, var_lp_f, con_ln, con_lp_f, nrm,
                     [eu['vc_W1'], r16(eu['vc_b1']), eu['vc_W2'], r16(eu['vc_b2']),
                      eu['cc_W1'], r16(eu['cc_b1']), eu['cc_W2'], r16(eu['cc_b2'])])
    vfg, cfg = _gather2(vf, src, cf, dst, 16, 16)
    outb, eln = _final_k(
        edge_learned_f, sc8, vfg, cfg, nrm,
        [eu['em_W1'], r16(eu['em_b1']), eu['em_W2'], r16(eu['em_b2']),
         ppw['W1'], ppw['b1'].reshape(1, 22), ppw['W2'], ppw['b2'].reshape(1, 1)])

    return (outb[:, 0], outb[:, 1], var_ln, con_ln, eln, outb[:, 2])
